# baseline (speedup 1.0000x reference)
import jax
import jax.numpy as jnp
from jax.experimental import pallas as pl
from jax.experimental.pallas import tpu as pltpu

CL = 4096
CF = 304
KG = 112
HID = 256
TAU = 0.8
BM = 512
NB = CL // BM


def _mlp_norm(z, w1, b1, w2, b2):
    h = jnp.dot(z.astype(jnp.bfloat16), w1.astype(jnp.bfloat16),
                preferred_element_type=jnp.float32) + b1
    h = jnp.where(h > 0, h, jnp.exp(h) - 1.0)
    p = jnp.dot(h.astype(jnp.bfloat16), w2.astype(jnp.bfloat16),
                preferred_element_type=jnp.float32) + b2
    inv_norm = jax.lax.rsqrt(jnp.sum(p * p, axis=1, keepdims=True))
    return p * inv_norm


def _proj2_kernel(z2_ref, w1_ref, b1_ref, w2_ref, b2_ref, out_ref):
    p2 = _mlp_norm(z2_ref[...], w1_ref[...], b1_ref[...],
                   w2_ref[...], b2_ref[...])
    out_ref[...] = p2.astype(jnp.bfloat16)


def _loss_kernel(z1_ref, p2_ref, p2diag_ref, w1cf_ref, b1cf_ref,
                 w2cf_ref, b2cf_ref, out_ref):
    p1 = _mlp_norm(z1_ref[...], w1cf_ref[...], b1cf_ref[...],
                   w2cf_ref[...], b2cf_ref[...]) * (1.0 / TAU)

    s = jax.lax.dot_general(
        p1.astype(jnp.bfloat16), p2_ref[...], (((1,), (1,)), ((), ())),
        preferred_element_type=jnp.float32,
    )
    rowsum = jnp.sum(jnp.exp(s), axis=1)

    diag = jnp.sum(p1 * p2diag_ref[...].astype(jnp.float32), axis=1)
    out_ref[0, 0, 0] = jnp.sum(jnp.log(rowsum + 1e-8) - diag) * (1.0 / CL)


def kernel(z1, z2, W1_cf, b1_cf, W2_cf, b2_cf, W1_kg, b1_kg, W2_kg, b2_kg):
    p2 = pl.pallas_call(
        _proj2_kernel,
        out_shape=jax.ShapeDtypeStruct((CL, HID), jnp.bfloat16),
    )(z2, W1_kg, b1_kg.reshape(1, HID), W2_kg, b2_kg.reshape(1, HID))

    const = lambda i: (0, 0)
    partials = pl.pallas_call(
        _loss_kernel,
        grid=(NB,),
        in_specs=[
            pl.BlockSpec((BM, CF), lambda i: (i, 0)),
            pl.BlockSpec((CL, HID), const),
            pl.BlockSpec((BM, HID), lambda i: (i, 0)),
            pl.BlockSpec((CF, HID), const),
            pl.BlockSpec((1, HID), const),
            pl.BlockSpec((HID, HID), const),
            pl.BlockSpec((1, HID), const),
        ],
        out_specs=pl.BlockSpec((1, 1, 1), lambda i: (i, 0, 0),
                               memory_space=pltpu.SMEM),
        out_shape=jax.ShapeDtypeStruct((NB, 1, 1), jnp.float32),
        compiler_params=pltpu.CompilerParams(
            dimension_semantics=("parallel",)),
    )(z1, p2, p2, W1_cf, b1_cf.reshape(1, HID), W2_cf, b2_cf.reshape(1, HID))
    return jnp.sum(partials)


# speedup vs baseline: 1.0617x; 1.0617x over previous
import jax
import jax.numpy as jnp
from jax.experimental import pallas as pl
from jax.experimental.pallas import tpu as pltpu

CL = 4096
CF = 304
KG = 112
HID = 256
TAU = 0.8
BM = 512
NB = CL // BM
CHUNK = 128


def _mlp_norm(z, w1, b1, w2, b2):
    h = jnp.dot(z.astype(jnp.bfloat16), w1.astype(jnp.bfloat16),
                preferred_element_type=jnp.float32) + b1
    h = jnp.where(h > 0, h, jnp.exp(h) - 1.0)
    p = jnp.dot(h.astype(jnp.bfloat16), w2.astype(jnp.bfloat16),
                preferred_element_type=jnp.float32) + b2
    inv_norm = jax.lax.rsqrt(jnp.sum(p * p, axis=1, keepdims=True))
    return p * inv_norm


def _fused_kernel(z1_ref, z2_ref, w1cf_ref, b1cf_ref, w2cf_ref, b2cf_ref,
                  w1kg_ref, b1kg_ref, w2kg_ref, b2kg_ref, out_ref, p2_scr):
    i = pl.program_id(0)

    @pl.when(i == 0)
    def _():
        p2 = _mlp_norm(z2_ref[...], w1kg_ref[...], b1kg_ref[...],
                       w2kg_ref[...], b2kg_ref[...])
        p2_scr[...] = p2.astype(jnp.bfloat16)

    p1 = _mlp_norm(z1_ref[...], w1cf_ref[...], b1cf_ref[...],
                   w2cf_ref[...], b2cf_ref[...]) * (1.0 / TAU)

    s = jax.lax.dot_general(
        p1.astype(jnp.bfloat16), p2_scr[...], (((1,), (1,)), ((), ())),
        preferred_element_type=jnp.float32,
    )
    acc = jnp.exp(s[:, :CHUNK])
    for k in range(1, CL // CHUNK):
        acc = acc + jnp.exp(s[:, k * CHUNK:(k + 1) * CHUNK])
    rowsum = jnp.sum(acc, axis=1)

    diag_blk = p2_scr[pl.ds(i * BM, BM), :].astype(jnp.float32)
    diag = jnp.sum(p1 * diag_blk, axis=1)
    partial = jnp.sum(jnp.log(rowsum + 1e-8) - diag) * (1.0 / CL)

    @pl.when(i == 0)
    def _():
        out_ref[0, 0] = 0.0

    out_ref[0, 0] += partial


def kernel(z1, z2, W1_cf, b1_cf, W2_cf, b2_cf, W1_kg, b1_kg, W2_kg, b2_kg):
    const = lambda i: (0, 0)
    out = pl.pallas_call(
        _fused_kernel,
        grid=(NB,),
        in_specs=[
            pl.BlockSpec((BM, CF), lambda i: (i, 0)),
            pl.BlockSpec((CL, KG), const),
            pl.BlockSpec((CF, HID), const),
            pl.BlockSpec((1, HID), const),
            pl.BlockSpec((HID, HID), const),
            pl.BlockSpec((1, HID), const),
            pl.BlockSpec((KG, HID), const),
            pl.BlockSpec((1, HID), const),
            pl.BlockSpec((HID, HID), const),
            pl.BlockSpec((1, HID), const),
        ],
        out_specs=pl.BlockSpec(memory_space=pltpu.SMEM),
        out_shape=jax.ShapeDtypeStruct((1, 1), jnp.float32),
        scratch_shapes=[pltpu.VMEM((CL, HID), jnp.bfloat16)],
    )(z1, z2, W1_cf, b1_cf.reshape(1, HID), W2_cf, b2_cf.reshape(1, HID),
      W1_kg, b1_kg.reshape(1, HID), W2_kg, b2_kg.reshape(1, HID))
    return out[0, 0]


# speedup vs baseline: 1.1621x; 1.0946x over previous
import jax
import jax.numpy as jnp
from jax.experimental import pallas as pl
from jax.experimental.pallas import tpu as pltpu

CL = 4096
CF = 304
KG = 112
HID = 256
TAU = 0.8
BM = 1024
NB = CL // BM
CHUNK = 128


def _mlp_norm(z, w1, b1, w2, b2):
    h = jnp.dot(z.astype(jnp.bfloat16), w1.astype(jnp.bfloat16),
                preferred_element_type=jnp.float32) + b1
    h = jnp.where(h > 0, h, jnp.exp(h) - 1.0)
    p = jnp.dot(h.astype(jnp.bfloat16), w2.astype(jnp.bfloat16),
                preferred_element_type=jnp.float32) + b2
    inv_norm = jax.lax.rsqrt(jnp.sum(p * p, axis=1, keepdims=True))
    return p * inv_norm


def _fused_kernel(z1_ref, z2_ref, w1cf_ref, b1cf_ref, w2cf_ref, b2cf_ref,
                  w1kg_ref, b1kg_ref, w2kg_ref, b2kg_ref, out_ref, p2_scr):
    i = pl.program_id(0)

    @pl.when(i == 0)
    def _():
        p2 = _mlp_norm(z2_ref[...], w1kg_ref[...], b1kg_ref[...],
                       w2kg_ref[...], b2kg_ref[...])
        p2_scr[...] = p2.astype(jnp.bfloat16)

    p1 = _mlp_norm(z1_ref[...], w1cf_ref[...], b1cf_ref[...],
                   w2cf_ref[...], b2cf_ref[...]) * (1.0 / TAU)

    s = jax.lax.dot_general(
        p1.astype(jnp.bfloat16), p2_scr[...], (((1,), (1,)), ((), ())),
        preferred_element_type=jnp.float32,
    )
    acc = jnp.exp(s[:, :CHUNK])
    for k in range(1, CL // CHUNK):
        acc = acc + jnp.exp(s[:, k * CHUNK:(k + 1) * CHUNK])
    rowsum = jnp.sum(acc, axis=1)

    diag_blk = p2_scr[pl.ds(i * BM, BM), :].astype(jnp.float32)
    diag = jnp.sum(p1 * diag_blk, axis=1)
    partial = jnp.sum(jnp.log(rowsum + 1e-8) - diag) * (1.0 / CL)

    @pl.when(i == 0)
    def _():
        out_ref[0, 0] = 0.0

    out_ref[0, 0] += partial


def kernel(z1, z2, W1_cf, b1_cf, W2_cf, b2_cf, W1_kg, b1_kg, W2_kg, b2_kg):
    const = lambda i: (0, 0)
    out = pl.pallas_call(
        _fused_kernel,
        grid=(NB,),
        in_specs=[
            pl.BlockSpec((BM, CF), lambda i: (i, 0)),
            pl.BlockSpec((CL, KG), const),
            pl.BlockSpec((CF, HID), const),
            pl.BlockSpec((1, HID), const),
            pl.BlockSpec((HID, HID), const),
            pl.BlockSpec((1, HID), const),
            pl.BlockSpec((KG, HID), const),
            pl.BlockSpec((1, HID), const),
            pl.BlockSpec((HID, HID), const),
            pl.BlockSpec((1, HID), const),
        ],
        out_specs=pl.BlockSpec(memory_space=pltpu.SMEM),
        out_shape=jax.ShapeDtypeStruct((1, 1), jnp.float32),
        scratch_shapes=[pltpu.VMEM((CL, HID), jnp.bfloat16)],
    )(z1, z2, W1_cf, b1_cf.reshape(1, HID), W2_cf, b2_cf.reshape(1, HID),
      W1_kg, b1_kg.reshape(1, HID), W2_kg, b2_kg.reshape(1, HID))
    return out[0, 0]
